# baseline (speedup 1.0000x reference)
import functools

import jax
import jax.numpy as jnp
from jax.experimental import pallas as pl
from jax.experimental.pallas import tpu as pltpu

_EPS = 1e-05
_B = 1000


def _mm_stats_kernel(x_ref, w_ref, y_ref, st_ref, acc_ref, *, nb):
    i = pl.program_id(0)

    @pl.when(i == 0)
    def _():
        acc_ref[...] = jnp.zeros_like(acc_ref)

    y = jnp.dot(x_ref[...], w_ref[...].T, preferred_element_type=jnp.float32)
    y_ref[...] = y
    acc_ref[0:1, :] += jnp.sum(y, axis=0, keepdims=True)
    acc_ref[1:2, :] += jnp.sum(y * y, axis=0, keepdims=True)

    @pl.when(i == nb - 1)
    def _():
        st_ref[...] = acc_ref[...]


def _matmul_stats(x, W):
    R, Din = x.shape
    Dout = W.shape[0]
    nb = R // _B
    return pl.pallas_call(
        functools.partial(_mm_stats_kernel, nb=nb),
        grid=(nb,),
        in_specs=[
            pl.BlockSpec((_B, Din), lambda i: (i, 0)),
            pl.BlockSpec((Dout, Din), lambda i: (0, 0)),
        ],
        out_specs=[
            pl.BlockSpec((_B, Dout), lambda i: (i, 0)),
            pl.BlockSpec((8, Dout), lambda i: (0, 0)),
        ],
        out_shape=[
            jax.ShapeDtypeStruct((R, Dout), jnp.float32),
            jax.ShapeDtypeStruct((8, Dout), jnp.float32),
        ],
        scratch_shapes=[pltpu.VMEM((8, Dout), jnp.float32)],
    )(x, W)


def _bnrelu_mm_kernel(y_ref, st_ref, w_ref, y2_ref, st2_ref, acc_ref, *, R, nb):
    i = pl.program_id(0)

    @pl.when(i == 0)
    def _():
        acc_ref[...] = jnp.zeros_like(acc_ref)

    mu = st_ref[0:1, :] / R
    var = st_ref[1:2, :] / R - mu * mu
    a = jnp.maximum((y_ref[...] - mu) * jax.lax.rsqrt(var + _EPS), 0.0)
    y2 = jnp.dot(a, w_ref[...].T, preferred_element_type=jnp.float32)
    y2_ref[...] = y2
    acc_ref[0:1, :] += jnp.sum(y2, axis=0, keepdims=True)
    acc_ref[1:2, :] += jnp.sum(y2 * y2, axis=0, keepdims=True)

    @pl.when(i == nb - 1)
    def _():
        st2_ref[...] = acc_ref[...]


def _bnrelu_mm_stats(y, st, W):
    R, Din = y.shape
    Dout = W.shape[0]
    nb = R // _B
    return pl.pallas_call(
        functools.partial(_bnrelu_mm_kernel, R=float(R), nb=nb),
        grid=(nb,),
        in_specs=[
            pl.BlockSpec((_B, Din), lambda i: (i, 0)),
            pl.BlockSpec((8, Din), lambda i: (0, 0)),
            pl.BlockSpec((Dout, Din), lambda i: (0, 0)),
        ],
        out_specs=[
            pl.BlockSpec((_B, Dout), lambda i: (i, 0)),
            pl.BlockSpec((8, Dout), lambda i: (0, 0)),
        ],
        out_shape=[
            jax.ShapeDtypeStruct((R, Dout), jnp.float32),
            jax.ShapeDtypeStruct((8, Dout), jnp.float32),
        ],
        scratch_shapes=[pltpu.VMEM((8, Dout), jnp.float32)],
    )(y, st, W)


def _bnrelu_kernel(y_ref, st_ref, o_ref, *, R):
    mu = st_ref[0:1, :] / R
    var = st_ref[1:2, :] / R - mu * mu
    o_ref[...] = jnp.maximum((y_ref[...] - mu) * jax.lax.rsqrt(var + _EPS), 0.0)


def _bnrelu(y, st):
    R, D = y.shape
    nb = R // _B
    return pl.pallas_call(
        functools.partial(_bnrelu_kernel, R=float(R)),
        grid=(nb,),
        in_specs=[
            pl.BlockSpec((_B, D), lambda i: (i, 0)),
            pl.BlockSpec((8, D), lambda i: (0, 0)),
        ],
        out_specs=pl.BlockSpec((_B, D), lambda i: (i, 0)),
        out_shape=jax.ShapeDtypeStruct((R, D), jnp.float32),
    )(y, st)


def _lin_bn_relu(x, W):
    y, st = _matmul_stats(x, W)
    return _bnrelu(y, st)


def _mlp2(x, Wa, Wb):
    y1, s1 = _matmul_stats(x, Wa)
    y2, s2 = _bnrelu_mm_stats(y1, s1, Wb)
    return _bnrelu(y2, s2)


def _seg_mean_bcast(x, seg, num):
    s = jax.ops.segment_sum(x, seg, num_segments=num)
    cnt = jax.ops.segment_sum(jnp.ones((x.shape[0], 1), x.dtype), seg,
                              num_segments=num)
    return (s / jnp.clip(cnt, 1.0))[seg]


def kernel(node_rep, edge_rep, cycle_rep, edge_index, cycle_entry_edge,
           cycle_domain, W_ne_lift1, W_ne_lift2, W_ne_lvl1, W_ne_lvl2a,
           W_ne_lvl2b, eps_ne_1, eps_ne_2, W_ec_lift1, W_ec_lift2, W_ec_lvl1,
           W_ec_lvl2a, W_ec_lvl2b, eps_ec_11, eps_ec_12, eps_ec_2, W_mlp):
    N = node_rep.shape[0]
    E = edge_rep.shape[0]
    C = 20000

    ei0, ei1 = edge_index[0], edge_index[1]

    lift_aggr = node_rep[ei0] + node_rep[ei1]
    h1 = _lin_bn_relu(jnp.concatenate([lift_aggr, edge_rep], -1), W_ne_lvl1)
    lvl_aggr = jnp.zeros((N, node_rep.shape[1]), node_rep.dtype)
    lvl_aggr = lvl_aggr.at[ei0].add(h1).at[ei1].add(h1)
    node_out = _mlp2((1.0 + eps_ne_1) * node_rep + lvl_aggr,
                     W_ne_lvl2a, W_ne_lvl2b)
    edge_out_1 = _mlp2((1.0 + eps_ne_2) * edge_rep + lift_aggr,
                       W_ne_lift1, W_ne_lift2)

    g = edge_rep[cycle_entry_edge]
    gm = _seg_mean_bcast(g, cycle_domain, C)
    lift_aggr2 = jnp.concatenate([g, gm], -1)
    h2 = _lin_bn_relu(jnp.concatenate([lift_aggr2, cycle_rep], -1), W_ec_lvl1)
    dm = _seg_mean_bcast(h2, cycle_domain, C)
    z = jax.ops.segment_sum((1.0 + eps_ec_12) * h2 + dm, cycle_entry_edge,
                            num_segments=E)
    edge_out_2 = _mlp2((1.0 + eps_ec_11) * edge_rep + z,
                       W_ec_lvl2a, W_ec_lvl2b)
    cyc_lin = jnp.concatenate(
        [cycle_rep, _seg_mean_bcast(cycle_rep, cycle_domain, C)], -1)
    cycle_out = _mlp2((1.0 + eps_ec_2) * cyc_lin + lift_aggr2,
                      W_ec_lift1, W_ec_lift2)

    edge_out = _lin_bn_relu(jnp.concatenate([edge_out_1, edge_out_2], -1),
                            W_mlp)
    return (node_out, edge_out, cycle_out)


# speedup vs baseline: 1.1063x; 1.1063x over previous
import functools

import jax
import jax.numpy as jnp
from jax.experimental import pallas as pl
from jax.experimental.pallas import tpu as pltpu

_EPS = 1e-05
_B = 1000


def _mm_groups_kernel(s_ref, *refs, meta, nparts, nb):
    parts = refs[:nparts]
    w_ref = refs[nparts]
    y_ref = refs[nparts + 1]
    st_ref = refs[nparts + 2]
    acc_ref = refs[nparts + 3]
    i = pl.program_id(0)

    @pl.when(i == 0)
    def _():
        acc_ref[...] = jnp.zeros_like(acc_ref)

    col = 0
    y = None
    for gm, width in meta:
        xg = None
        for pidx, sidx in gm:
            v = parts[pidx][...]
            if sidx is not None:
                v = v * s_ref[sidx]
            xg = v if xg is None else xg + v
        contrib = jnp.dot(xg, w_ref[:, col:col + width].T,
                          preferred_element_type=jnp.float32)
        y = contrib if y is None else y + contrib
        col += width
    y_ref[...] = y
    acc_ref[0:1, :] += jnp.sum(y, axis=0, keepdims=True)
    acc_ref[1:2, :] += jnp.sum(y * y, axis=0, keepdims=True)

    @pl.when(i == nb - 1)
    def _():
        st_ref[...] = acc_ref[...]


def _fused_mm_stats(groups, W):
    parts, meta, svals = [], [], [jnp.float32(1.0)]
    for g in groups:
        gm = []
        width = g[0][0].shape[1]
        for a, s in g:
            pidx = len(parts)
            parts.append(a)
            sidx = None
            if s is not None:
                sidx = len(svals)
                svals.append(jnp.asarray(s, jnp.float32))
            gm.append((pidx, sidx))
        meta.append((tuple(gm), width))
    svec = jnp.stack(svals)
    R = parts[0].shape[0]
    Dout, Din = W.shape
    nb = R // _B
    in_specs = [pl.BlockSpec(memory_space=pltpu.SMEM)]
    in_specs += [pl.BlockSpec((_B, p.shape[1]), lambda i: (i, 0))
                 for p in parts]
    in_specs += [pl.BlockSpec((Dout, Din), lambda i: (0, 0))]
    return pl.pallas_call(
        functools.partial(_mm_groups_kernel, meta=tuple(meta),
                          nparts=len(parts), nb=nb),
        grid=(nb,),
        in_specs=in_specs,
        out_specs=[
            pl.BlockSpec((_B, Dout), lambda i: (i, 0)),
            pl.BlockSpec((8, Dout), lambda i: (0, 0)),
        ],
        out_shape=[
            jax.ShapeDtypeStruct((R, Dout), jnp.float32),
            jax.ShapeDtypeStruct((8, Dout), jnp.float32),
        ],
        scratch_shapes=[pltpu.VMEM((8, Dout), jnp.float32)],
    )(svec, *parts, W)


def _bnrelu_mm_kernel(*refs, R, nb, has_add, with_stats):
    y_ref, st_ref, w_ref = refs[0], refs[1], refs[2]
    k = 3
    add_ref = None
    if has_add:
        add_ref = refs[k]
        k += 1
    y2_ref = refs[k]
    k += 1
    i = pl.program_id(0)
    mu = st_ref[0:1, :] / R
    var = st_ref[1:2, :] / R - mu * mu
    a = jnp.maximum((y_ref[...] - mu) * jax.lax.rsqrt(var + _EPS), 0.0)
    y2 = jnp.dot(a, w_ref[...].T, preferred_element_type=jnp.float32)
    if has_add:
        y2 = y2 + add_ref[...]
    y2_ref[...] = y2
    if with_stats:
        st2_ref, acc_ref = refs[k], refs[k + 1]

        @pl.when(i == 0)
        def _():
            acc_ref[...] = jnp.zeros_like(acc_ref)

        acc_ref[0:1, :] += jnp.sum(y2, axis=0, keepdims=True)
        acc_ref[1:2, :] += jnp.sum(y2 * y2, axis=0, keepdims=True)

        @pl.when(i == nb - 1)
        def _():
            st2_ref[...] = acc_ref[...]


def _bnrelu_mm(y, st, W, add=None, with_stats=True):
    R, Din = y.shape
    Dout = W.shape[0]
    nb = R // _B
    in_specs = [
        pl.BlockSpec((_B, Din), lambda i: (i, 0)),
        pl.BlockSpec((8, Din), lambda i: (0, 0)),
        pl.BlockSpec((Dout, Din), lambda i: (0, 0)),
    ]
    args = [y, st, W]
    if add is not None:
        in_specs.append(pl.BlockSpec((_B, Dout), lambda i: (i, 0)))
        args.append(add)
    out_specs = [pl.BlockSpec((_B, Dout), lambda i: (i, 0))]
    out_shape = [jax.ShapeDtypeStruct((R, Dout), jnp.float32)]
    scratch = []
    if with_stats:
        out_specs.append(pl.BlockSpec((8, Dout), lambda i: (0, 0)))
        out_shape.append(jax.ShapeDtypeStruct((8, Dout), jnp.float32))
        scratch.append(pltpu.VMEM((8, Dout), jnp.float32))
    res = pl.pallas_call(
        functools.partial(_bnrelu_mm_kernel, R=float(R), nb=nb,
                          has_add=add is not None, with_stats=with_stats),
        grid=(nb,),
        in_specs=in_specs,
        out_specs=out_specs,
        out_shape=out_shape,
        scratch_shapes=scratch,
    )(*args)
    return res if with_stats else res[0]


def _bnrelu_kernel(y_ref, st_ref, o_ref, *, R):
    mu = st_ref[0:1, :] / R
    var = st_ref[1:2, :] / R - mu * mu
    o_ref[...] = jnp.maximum((y_ref[...] - mu) * jax.lax.rsqrt(var + _EPS), 0.0)


def _bnrelu(y, st):
    R, D = y.shape
    nb = R // _B
    return pl.pallas_call(
        functools.partial(_bnrelu_kernel, R=float(R)),
        grid=(nb,),
        in_specs=[
            pl.BlockSpec((_B, D), lambda i: (i, 0)),
            pl.BlockSpec((8, D), lambda i: (0, 0)),
        ],
        out_specs=pl.BlockSpec((_B, D), lambda i: (i, 0)),
        out_shape=jax.ShapeDtypeStruct((R, D), jnp.float32),
    )(y, st)


def _seg_mean_bcast(x, seg, num):
    s = jax.ops.segment_sum(x, seg, num_segments=num)
    cnt = jax.ops.segment_sum(jnp.ones((x.shape[0], 1), x.dtype), seg,
                              num_segments=num)
    return (s / jnp.clip(cnt, 1.0))[seg]


def kernel(node_rep, edge_rep, cycle_rep, edge_index, cycle_entry_edge,
           cycle_domain, W_ne_lift1, W_ne_lift2, W_ne_lvl1, W_ne_lvl2a,
           W_ne_lvl2b, eps_ne_1, eps_ne_2, W_ec_lift1, W_ec_lift2, W_ec_lvl1,
           W_ec_lvl2a, W_ec_lvl2b, eps_ec_11, eps_ec_12, eps_ec_2, W_mlp):
    N, H = node_rep.shape
    E = edge_rep.shape[0]
    C = 20000

    ei0, ei1 = edge_index[0], edge_index[1]

    lift_aggr = node_rep[ei0] + node_rep[ei1]
    y, st = _fused_mm_stats([[(lift_aggr, None)], [(edge_rep, None)]],
                            W_ne_lvl1)
    h1 = _bnrelu(y, st)
    lvl_aggr = jnp.zeros((N, H), node_rep.dtype)
    lvl_aggr = lvl_aggr.at[ei0].add(h1).at[ei1].add(h1)

    y1, s1 = _fused_mm_stats(
        [[(node_rep, 1.0 + eps_ne_1), (lvl_aggr, None)]], W_ne_lvl2a)
    y2, s2 = _bnrelu_mm(y1, s1, W_ne_lvl2b)
    node_out = _bnrelu(y2, s2)

    y1, s1 = _fused_mm_stats(
        [[(edge_rep, 1.0 + eps_ne_2), (lift_aggr, None)]], W_ne_lift1)
    y2, s2 = _bnrelu_mm(y1, s1, W_ne_lift2)
    p1 = _bnrelu_mm(y2, s2, W_mlp[:, :H], with_stats=False)

    g = edge_rep[cycle_entry_edge]
    gm = _seg_mean_bcast(g, cycle_domain, C)
    y, st = _fused_mm_stats(
        [[(g, None)], [(gm, None)], [(cycle_rep, None)]], W_ec_lvl1)
    h2 = _bnrelu(y, st)
    dm = _seg_mean_bcast(h2, cycle_domain, C)
    z = jax.ops.segment_sum((1.0 + eps_ec_12) * h2 + dm, cycle_entry_edge,
                            num_segments=E)

    y1, s1 = _fused_mm_stats(
        [[(edge_rep, 1.0 + eps_ec_11), (z, None)]], W_ec_lvl2a)
    y2, s2 = _bnrelu_mm(y1, s1, W_ec_lvl2b)
    yf, sf = _bnrelu_mm(y2, s2, W_mlp[:, H:], add=p1)
    edge_out = _bnrelu(yf, sf)

    cm = _seg_mean_bcast(cycle_rep, cycle_domain, C)
    y1, s1 = _fused_mm_stats(
        [[(cycle_rep, 1.0 + eps_ec_2), (g, None)],
         [(cm, 1.0 + eps_ec_2), (gm, None)]], W_ec_lift1)
    y2, s2 = _bnrelu_mm(y1, s1, W_ec_lift2)
    cycle_out = _bnrelu(y2, s2)

    return (node_out, edge_out, cycle_out)
